# baseline (speedup 1.0000x reference)
import jax
import jax.numpy as jnp
from jax import lax
from jax.experimental import pallas as pl
from jax.experimental.pallas import tpu as pltpu
from jax.experimental.pallas import tpu_sc as plsc

R, N = 64, 32768
L = 16
NC, NS = 2, 16
NW = NC * NS
ROWS_PER_W = R // NW
U = 8
STEPS = N // (U * L)

_INT_MAX = 2**31 - 1


def _tec_body(x_hbm, out_hbm, buf, res_ref):
    c = lax.axis_index("c")
    s = lax.axis_index("s")
    w = s * NC + c
    lane = lax.iota(jnp.int32, L)
    res = jnp.zeros((L,), jnp.int32)

    for r in range(ROWS_PER_W):
        row = w * ROWS_PER_W + r
        pltpu.sync_copy(x_hbm.at[row], buf)

        def step(t, carry):
            bvs, bts = carry
            new_bvs, new_bts = [], []
            t_vec = jnp.zeros((L,), jnp.int32) + t
            for k in range(U):
                v = buf[pl.ds(t * (U * L) + k * L, L)]
                m = v > bvs[k]
                new_bvs.append(jnp.where(m, v, bvs[k]))
                new_bts.append(jnp.where(m, t_vec, bts[k]))
            return tuple(new_bvs), tuple(new_bts)

        bv0 = tuple(jnp.full((L,), -jnp.inf, jnp.float32) for _ in range(U))
        bt0 = tuple(jnp.zeros((L,), jnp.int32) for _ in range(U))
        bvs, bts = lax.fori_loop(0, STEPS, step, (bv0, bt0))

        pairs = [
            (bvs[k], bts[k] * (U * L) + (k * L) + lane) for k in range(U)
        ]
        while len(pairs) > 1:
            nxt = []
            for a in range(0, len(pairs), 2):
                (va, ia), (vb, ib) = pairs[a], pairs[a + 1]
                take_b = (vb > va) | ((vb == va) & (ib < ia))
                nxt.append((jnp.where(take_b, vb, va),
                            jnp.where(take_b, ib, ia)))
            pairs = nxt
        bv, bi = pairs[0]

        for d in (1, 2, 4, 8):
            perm = lane ^ d
            pv = bv.at[perm].get(mode="promise_in_bounds")
            pi = bi.at[perm].get(mode="promise_in_bounds")
            take_p = (pv > bv) | ((pv == bv) & (pi < bi))
            bv = jnp.where(take_p, pv, bv)
            bi = jnp.where(take_p, pi, bi)
        res = jnp.where(lane == r, bi, res)

    res_ref[...] = res
    pltpu.sync_copy(res_ref, out_hbm.at[w])


@jax.jit
def _argmax_rows(x):
    mesh = plsc.VectorSubcoreMesh(
        core_axis_name="c", subcore_axis_name="s",
        num_cores=NC, num_subcores=NS,
    )
    padded = pl.kernel(
        _tec_body,
        out_type=jax.ShapeDtypeStruct((NW, L), jnp.int32),
        mesh=mesh,
        scratch_types=[
            pltpu.VMEM((N,), jnp.float32),
            pltpu.VMEM((L,), jnp.int32),
        ],
    )(x)
    return padded[:, :ROWS_PER_W].reshape(R)


def kernel(x):
    return _argmax_rows(x)


# speedup vs baseline: 3.6701x; 3.6701x over previous
import jax
import jax.numpy as jnp
from jax import lax
from jax.experimental import pallas as pl
from jax.experimental.pallas import tpu as pltpu

R, N = 64, 32768
LANES = 128
BN = 4096
GRID = N // BN
CHUNKS = BN // LANES

_INT_MAX = 2**31 - 1


def _tc_body(x_ref, o_ref, rv_ref, ri_ref):
    i = pl.program_id(0)

    @pl.when(i == 0)
    def _init():
        rv_ref[...] = jnp.full((R, LANES), -jnp.inf, jnp.float32)
        ri_ref[...] = jnp.zeros((R, LANES), jnp.int32)

    rv = rv_ref[...]
    ri = ri_ref[...]
    for jj in range(CHUNKS):
        chunk = x_ref[:, jj * LANES:(jj + 1) * LANES]
        m = chunk > rv
        rv = jnp.where(m, chunk, rv)
        ri = jnp.where(m, i * CHUNKS + jj, ri)
    rv_ref[...] = rv
    ri_ref[...] = ri

    @pl.when(i == GRID - 1)
    def _finish():
        lane = lax.broadcasted_iota(jnp.int32, (R, LANES), 1)
        idx = ri * LANES + lane
        mx = jnp.max(rv, axis=1, keepdims=True)
        cand = jnp.where(rv == mx, idx, _INT_MAX)
        o_ref[...] = jnp.min(cand, axis=1)[None, :]


@jax.jit
def _argmax_rows(x):
    out = pl.pallas_call(
        _tc_body,
        grid=(GRID,),
        in_specs=[pl.BlockSpec((R, BN), lambda i: (0, i))],
        out_specs=pl.BlockSpec((1, R), lambda i: (0, 0)),
        out_shape=jax.ShapeDtypeStruct((1, R), jnp.int32),
        scratch_shapes=[
            pltpu.VMEM((R, LANES), jnp.float32),
            pltpu.VMEM((R, LANES), jnp.int32),
        ],
    )(x)
    return out.reshape(R)


def kernel(x):
    return _argmax_rows(x)
